# Initial kernel scaffold; baseline (speedup 1.0000x reference)
import jax
import jax.numpy as jnp
from jax.experimental import pallas as pl


def kernel(hidden_states, gate_w, w1, w3, w2):
    raise NotImplementedError("write your pallas kernel here")


# speedup vs baseline: 1.3296x; 1.3296x over previous
import jax
import jax.numpy as jnp
from jax.experimental import pallas as pl
from jax.experimental.pallas import tpu as pltpu

HID = 1024
FFN = 3584
E = 8
T = 128
FB = 512
NFB = FFN // FB


def _moe_body(x_ref, gw_ref, w1_ref, w3_ref, w2_ref, out_ref):
    e = pl.program_id(0)
    f = pl.program_id(1)
    x = x_ref[...]

    logits = jax.lax.dot_general(
        x, gw_ref[...], (((1,), (1,)), ((), ())))
    iota = jax.lax.broadcasted_iota(jnp.int32, (T, E), 1)
    v1 = jnp.max(logits, axis=1, keepdims=True)
    i1 = jnp.min(jnp.where(logits == v1, iota, E), axis=1, keepdims=True)
    masked = jnp.where(iota == i1, -jnp.inf, logits)
    v2 = jnp.max(masked, axis=1, keepdims=True)
    i2 = jnp.min(jnp.where(masked == v2, iota, E), axis=1, keepdims=True)
    p1 = jax.nn.sigmoid(v1 - v2)
    combine = jnp.where(i1 == e, p1, jnp.where(i2 == e, 1.0 - p1, 0.0))

    w1b = w1_ref[0]
    w3b = w3_ref[0]
    w2b = w2_ref[0]
    h = jax.lax.dot_general(x, w1b, (((1,), (1,)), ((), ())))
    g = jax.lax.dot_general(x, w3b, (((1,), (1,)), ((), ())))
    act = (h * jax.nn.sigmoid(h)) * g
    act = act * combine
    outp = jax.lax.dot_general(act, w2b, (((1,), (1,)), ((), ())))

    @pl.when(jnp.logical_and(e == 0, f == 0))
    def _init():
        out_ref[...] = jnp.zeros_like(out_ref)

    out_ref[...] += outp


def kernel(hidden_states, gate_w, w1, w3, w2):
    return pl.pallas_call(
        _moe_body,
        grid=(E, NFB),
        in_specs=[
            pl.BlockSpec((T, HID), lambda e, f: (0, 0)),
            pl.BlockSpec((E, HID), lambda e, f: (0, 0)),
            pl.BlockSpec((1, FB, HID), lambda e, f: (e, f, 0)),
            pl.BlockSpec((1, FB, HID), lambda e, f: (e, f, 0)),
            pl.BlockSpec((1, HID, FB), lambda e, f: (e, 0, f)),
        ],
        out_specs=pl.BlockSpec((T, HID), lambda e, f: (0, 0)),
        out_shape=jax.ShapeDtypeStruct((T, HID), hidden_states.dtype),
        compiler_params=pltpu.CompilerParams(
            dimension_semantics=("arbitrary", "arbitrary"),
        ),
    )(hidden_states, gate_w, w1, w3, w2)


# speedup vs baseline: 1.5266x; 1.1481x over previous
import jax
import jax.numpy as jnp
from jax.experimental import pallas as pl
from jax.experimental.pallas import tpu as pltpu

HID = 1024
FFN = 3584
E = 8
T = 128
FB = 896
NFB = FFN // FB


def _moe_body(x_ref, gw_ref, w1_ref, w3_ref, w2_ref, out_ref):
    e = pl.program_id(0)
    f = pl.program_id(1)
    x = x_ref[...]

    logits = jax.lax.dot_general(
        x, gw_ref[...], (((1,), (1,)), ((), ())))
    iota = jax.lax.broadcasted_iota(jnp.int32, (T, E), 1)
    v1 = jnp.max(logits, axis=1, keepdims=True)
    i1 = jnp.min(jnp.where(logits == v1, iota, E), axis=1, keepdims=True)
    masked = jnp.where(iota == i1, -jnp.inf, logits)
    v2 = jnp.max(masked, axis=1, keepdims=True)
    i2 = jnp.min(jnp.where(masked == v2, iota, E), axis=1, keepdims=True)
    p1 = jax.nn.sigmoid(v1 - v2)
    combine = jnp.where(i1 == e, p1, jnp.where(i2 == e, 1.0 - p1, 0.0))

    w1b = w1_ref[0]
    w3b = w3_ref[0]
    w2b = w2_ref[0]
    h = jax.lax.dot_general(x, w1b, (((1,), (1,)), ((), ())))
    g = jax.lax.dot_general(x, w3b, (((1,), (1,)), ((), ())))
    act = (h * jax.nn.sigmoid(h)) * g
    act = act * combine
    outp = jax.lax.dot_general(act, w2b, (((1,), (1,)), ((), ())))

    @pl.when(jnp.logical_and(e == 0, f == 0))
    def _init():
        out_ref[...] = jnp.zeros_like(out_ref)

    out_ref[...] += outp


def kernel(hidden_states, gate_w, w1, w3, w2):
    return pl.pallas_call(
        _moe_body,
        grid=(E, NFB),
        in_specs=[
            pl.BlockSpec((T, HID), lambda e, f: (0, 0)),
            pl.BlockSpec((E, HID), lambda e, f: (0, 0)),
            pl.BlockSpec((1, FB, HID), lambda e, f: (e, f, 0)),
            pl.BlockSpec((1, FB, HID), lambda e, f: (e, f, 0)),
            pl.BlockSpec((1, HID, FB), lambda e, f: (e, 0, f)),
        ],
        out_specs=pl.BlockSpec((T, HID), lambda e, f: (0, 0)),
        out_shape=jax.ShapeDtypeStruct((T, HID), hidden_states.dtype),
        compiler_params=pltpu.CompilerParams(
            dimension_semantics=("arbitrary", "arbitrary"),
        ),
    )(hidden_states, gate_w, w1, w3, w2)
